# baseline (speedup 1.0000x reference)
import functools

import jax
import jax.numpy as jnp
from jax import lax
from jax.experimental import pallas as pl
from jax.experimental.pallas import tpu as pltpu
from jax.experimental.pallas import tpu_sc as plsc

N = 10000
E = 320000
D = 128
NC = 2
NS = 16
L = 16
NW = NC * NS
EPW = E // NW
NPAD = 10240
RPT = NPAD // NS
CHW = 128
NCHW = 80
NPAIR = NCHW // 2
EPAD = NW * NCHW * CHW

_mesh = plsc.VectorSubcoreMesh(core_axis_name="c", subcore_axis_name="s")
_sc_params = pltpu.CompilerParams(needs_layout_passes=False)


def _zero_ref(ref, nwords):
    z = jnp.zeros((L,), jnp.float32)

    def body(i, c):
        ref[pl.ds(i * L, L)] = z
        return c

    lax.fori_loop(0, nwords // L, body, 0)


def _reduce_tiles(shared, red_v, tmp_v, out_hbm, cid, sid):
    base = sid * RPT
    pltpu.sync_copy(shared.at[0, pl.ds(base, RPT)], red_v)

    def body(k, c):
        pltpu.sync_copy(shared.at[k, pl.ds(base, RPT)], tmp_v)

        def add(i, c2):
            red_v[pl.ds(i * L, L)] = red_v[pl.ds(i * L, L)] + tmp_v[pl.ds(i * L, L)]
            return c2

        lax.fori_loop(0, RPT // L, add, 0)
        return c

    lax.fori_loop(1, NS, body, 0)
    pltpu.sync_copy(red_v, out_hbm.at[cid, pl.ds(base, RPT)])


@functools.partial(
    pl.kernel,
    out_type=jax.ShapeDtypeStruct((NC, NPAD), jnp.float32),
    mesh=_mesh,
    compiler_params=_sc_params,
    scratch_types=[
        pltpu.VMEM((EPW,), jnp.int32),
        pltpu.VMEM((NPAD,), jnp.float32),
        pltpu.VMEM((RPT,), jnp.float32),
        pltpu.VMEM((RPT,), jnp.float32),
        pltpu.VMEM_SHARED((NS, NPAD), jnp.float32),
    ],
)
def _deg_kernel(dst_hbm, out_hbm, idx_v, acc_v, red_v, tmp_v, shared):
    cid = lax.axis_index("c")
    sid = lax.axis_index("s")
    wid = cid * NS + sid
    _zero_ref(acc_v, NPAD)
    pltpu.sync_copy(dst_hbm.at[pl.ds(wid * EPW, EPW)], idx_v)
    ones = jnp.ones((L,), jnp.float32)

    def body(i, c):
        dvec = idx_v[pl.ds(i * L, L)]
        plsc.addupdate_scatter(acc_v, [dvec], ones)
        return c

    lax.fori_loop(0, EPW // L, body, 0)
    pltpu.sync_copy(acc_v, shared.at[sid])
    plsc.subcore_barrier()
    _reduce_tiles(shared, red_v, tmp_v, out_hbm, cid, sid)


@functools.partial(
    pl.kernel,
    out_type=jax.ShapeDtypeStruct((NC, NPAD, D), jnp.float32),
    mesh=_mesh,
    compiler_params=_sc_params,
    scratch_types=[
        pltpu.VMEM((2, CHW), jnp.int32),
        pltpu.VMEM((NCHW, CHW), jnp.int32),
        pltpu.VMEM((2, CHW, D), jnp.float32),
        pltpu.SemaphoreType.DMA,
        pltpu.SemaphoreType.DMA,
        pltpu.VMEM_SHARED((NPAD, D), jnp.float32),
    ],
)
def _row_agg_kernel(hs_hbm, src3_hbm, dst3_hbm, zero_hbm, out_hbm,
                    sring_v, didx_v, rows_v, sem_a, sem_b, acc_sh):
    cid = lax.axis_index("c")
    sid = lax.axis_index("s")
    wid = cid * NS + sid
    pltpu.sync_copy(zero_hbm.at[pl.ds(sid * RPT, RPT)],
                    acc_sh.at[pl.ds(sid * RPT, RPT)])
    pltpu.sync_copy(dst3_hbm.at[wid], didx_v)
    plsc.subcore_barrier()

    def g(b, sem):
        return pltpu.make_async_copy(hs_hbm.at[sring_v.at[b]],
                                     rows_v.at[b], sem)

    pltpu.sync_copy(src3_hbm.at[wid, 0], sring_v.at[0])
    g(0, sem_a).start()

    def body(p, c):
        c0 = 2 * p
        pltpu.sync_copy(src3_hbm.at[wid, c0 + 1], sring_v.at[1])
        g(1, sem_b).start()
        g(0, sem_a).wait()
        pltpu.sync_copy(rows_v.at[0], acc_sh.at[didx_v.at[c0]], add=True)

        @pl.when(p < NPAIR - 1)
        def _():
            pltpu.sync_copy(src3_hbm.at[wid, c0 + 2], sring_v.at[0])
            g(0, sem_a).start()

        g(1, sem_b).wait()
        pltpu.sync_copy(rows_v.at[1], acc_sh.at[didx_v.at[c0 + 1]], add=True)
        return c

    lax.fori_loop(0, NPAIR, body, 0)
    plsc.subcore_barrier()
    pltpu.sync_copy(acc_sh.at[pl.ds(sid * RPT, RPT)],
                    out_hbm.at[cid, pl.ds(sid * RPT, RPT)])


@functools.partial(
    pl.kernel,
    out_type=jax.ShapeDtypeStruct((NC, NPAD), jnp.float32),
    mesh=_mesh,
    compiler_params=_sc_params,
    scratch_types=[
        pltpu.VMEM((EPW,), jnp.int32),
        pltpu.VMEM((EPW,), jnp.int32),
        pltpu.VMEM((N,), jnp.float32),
        pltpu.VMEM((NPAD,), jnp.float32),
        pltpu.VMEM((RPT,), jnp.float32),
        pltpu.VMEM((RPT,), jnp.float32),
        pltpu.VMEM_SHARED((NS, NPAD), jnp.float32),
    ],
)
def _scalar_agg_kernel(ss_hbm, src_hbm, dst_hbm, out_hbm,
                       sidx_v, didx_v, ss_v, acc_v, red_v, tmp_v, shared):
    cid = lax.axis_index("c")
    sid = lax.axis_index("s")
    wid = cid * NS + sid
    _zero_ref(acc_v, NPAD)
    pltpu.sync_copy(ss_hbm, ss_v)
    pltpu.sync_copy(src_hbm.at[pl.ds(wid * EPW, EPW)], sidx_v)
    pltpu.sync_copy(dst_hbm.at[pl.ds(wid * EPW, EPW)], didx_v)

    def body(i, c):
        svec = sidx_v[pl.ds(i * L, L)]
        dvec = didx_v[pl.ds(i * L, L)]
        vals = plsc.load_gather(ss_v, [svec])
        plsc.addupdate_scatter(acc_v, [dvec], vals)
        return c

    lax.fori_loop(0, EPW // L, body, 0)
    pltpu.sync_copy(acc_v, shared.at[sid])
    plsc.subcore_barrier()
    _reduce_tiles(shared, red_v, tmp_v, out_hbm, cid, sid)


_BR = 1000


def _tc_hs_body(x_ref, w_ref, degp_ref, hs_ref):
    deg = degp_ref[0] + degp_ref[1] + 1.0
    dinv = lax.rsqrt(deg)
    h = jnp.dot(jnp.abs(x_ref[...]), w_ref[...],
                preferred_element_type=jnp.float32)
    hs_ref[...] = h * dinv


def _tc_mid_body(accp_ref, hs_ref, degp_ref, b1_ref, w2_ref, ss_ref):
    deg = degp_ref[0] + degp_ref[1] + 1.0
    dinv = lax.rsqrt(deg)
    tot = accp_ref[0] + accp_ref[1] + hs_ref[...]
    out1 = jnp.maximum(tot * dinv + b1_ref[...], 0.0)
    s = jnp.dot(out1, w2_ref[...], preferred_element_type=jnp.float32)
    ss_ref[...] = s * dinv


def _tc_out_body(qp_ref, ss_ref, degp_ref, b2_ref, o_ref):
    deg = degp_ref[0] + degp_ref[1] + 1.0
    dinv = lax.rsqrt(deg)
    out2 = (qp_ref[0] + qp_ref[1] + ss_ref[...]) * dinv + b2_ref[...]
    o_ref[...] = jax.nn.sigmoid(jnp.maximum(out2, 0.0))


def kernel(x, edge_index, W1, b1, W2, b2):
    src = edge_index[0].astype(jnp.int32)
    dst = edge_index[1].astype(jnp.int32)

    degp = _deg_kernel(dst)
    degp3 = degp[:, :N].reshape(NC, N, 1)

    hs = pl.pallas_call(
        _tc_hs_body,
        grid=(N // _BR,),
        in_specs=[
            pl.BlockSpec((_BR, D), lambda i: (i, 0)),
            pl.BlockSpec((D, D), lambda i: (0, 0)),
            pl.BlockSpec((NC, _BR, 1), lambda i: (0, i, 0)),
        ],
        out_specs=pl.BlockSpec((_BR, D), lambda i: (i, 0)),
        out_shape=jax.ShapeDtypeStruct((N, D), jnp.float32),
    )(x, W1, degp3)

    zrows = jnp.zeros((NPAD, D), jnp.float32)
    srcp = jnp.concatenate([src, jnp.zeros((EPAD - E,), jnp.int32)])
    dstp = jnp.concatenate([dst, jnp.full((EPAD - E,), NPAD - 1, jnp.int32)])
    accp = _row_agg_kernel(hs, srcp.reshape(NW, NCHW, CHW),
                           dstp.reshape(NW, NCHW, CHW), zrows)
    accp_n = accp[:, :N, :]

    ss = pl.pallas_call(
        _tc_mid_body,
        grid=(N // _BR,),
        in_specs=[
            pl.BlockSpec((NC, _BR, D), lambda i: (0, i, 0)),
            pl.BlockSpec((_BR, D), lambda i: (i, 0)),
            pl.BlockSpec((NC, _BR, 1), lambda i: (0, i, 0)),
            pl.BlockSpec((1, D), lambda i: (0, 0)),
            pl.BlockSpec((D, 1), lambda i: (0, 0)),
        ],
        out_specs=pl.BlockSpec((_BR, 1), lambda i: (i, 0)),
        out_shape=jax.ShapeDtypeStruct((N, 1), jnp.float32),
    )(accp_n, hs, degp3, b1.reshape(1, D), W2)

    qp = _scalar_agg_kernel(ss.reshape(N), src, dst)
    qp3 = qp[:, :N].reshape(NC, N, 1)

    y = pl.pallas_call(
        _tc_out_body,
        grid=(1,),
        in_specs=[
            pl.BlockSpec((NC, N, 1), lambda i: (0, 0, 0)),
            pl.BlockSpec((N, 1), lambda i: (0, 0)),
            pl.BlockSpec((NC, N, 1), lambda i: (0, 0, 0)),
            pl.BlockSpec((1, 1), lambda i: (0, 0)),
        ],
        out_specs=pl.BlockSpec((N, 1), lambda i: (0, 0)),
        out_shape=jax.ShapeDtypeStruct((N, 1), jnp.float32),
    )(qp3, ss, degp3, b2.reshape(1, 1))

    return y


# speedup vs baseline: 2.5324x; 2.5324x over previous
import functools

import jax
import jax.numpy as jnp
from jax import lax
from jax.experimental import pallas as pl
from jax.experimental.pallas import tpu as pltpu
from jax.experimental.pallas import tpu_sc as plsc

N = 10000
E = 320000
D = 128
NC = 2
NS = 16
L = 16
NW = NC * NS
EPW = E // NW
NPAD = 10240
RPT = NPAD // NS
CHW = 128
NCHW = 80
NPAIR = NCHW // 2
EPAD = NW * NCHW * CHW

_mesh = plsc.VectorSubcoreMesh(core_axis_name="c", subcore_axis_name="s")
_sc_params = pltpu.CompilerParams(needs_layout_passes=False)


def _zero_ref(ref, nwords):
    z = jnp.zeros((L,), jnp.float32)

    def body(i, c):
        ref[pl.ds(i * L, L)] = z
        return c

    lax.fori_loop(0, nwords // L, body, 0)


def _reduce_tiles(shared, red_v, tmp_v, out_hbm, cid, sid):
    base = sid * RPT
    pltpu.sync_copy(shared.at[0, pl.ds(base, RPT)], red_v)

    def body(k, c):
        pltpu.sync_copy(shared.at[k, pl.ds(base, RPT)], tmp_v)

        def add(i, c2):
            red_v[pl.ds(i * L, L)] = red_v[pl.ds(i * L, L)] + tmp_v[pl.ds(i * L, L)]
            return c2

        lax.fori_loop(0, RPT // L, add, 0)
        return c

    lax.fori_loop(1, NS, body, 0)
    pltpu.sync_copy(red_v, out_hbm.at[cid, pl.ds(base, RPT)])


@functools.partial(
    pl.kernel,
    out_type=jax.ShapeDtypeStruct((NC, NPAD), jnp.float32),
    mesh=_mesh,
    compiler_params=_sc_params,
    scratch_types=[
        pltpu.VMEM((EPW,), jnp.int32),
        pltpu.VMEM((NPAD,), jnp.float32),
        pltpu.VMEM((RPT,), jnp.float32),
        pltpu.VMEM((RPT,), jnp.float32),
        pltpu.VMEM_SHARED((NS, NPAD), jnp.float32),
    ],
)
def _deg_kernel(dst_hbm, out_hbm, idx_v, acc_v, red_v, tmp_v, shared):
    cid = lax.axis_index("c")
    sid = lax.axis_index("s")
    wid = cid * NS + sid
    _zero_ref(acc_v, NPAD)
    pltpu.sync_copy(dst_hbm.at[pl.ds(wid * EPW, EPW)], idx_v)
    ones = jnp.ones((L,), jnp.float32)

    def body(i, c):
        dvec = idx_v[pl.ds(i * L, L)]
        plsc.addupdate_scatter(acc_v, [dvec], ones)
        return c

    lax.fori_loop(0, EPW // L, body, 0)
    pltpu.sync_copy(acc_v, shared.at[sid])
    plsc.subcore_barrier()
    _reduce_tiles(shared, red_v, tmp_v, out_hbm, cid, sid)


@functools.partial(
    pl.kernel,
    out_type=jax.ShapeDtypeStruct((NC, NPAD, D), jnp.float32),
    mesh=_mesh,
    compiler_params=_sc_params,
    scratch_types=[
        pltpu.VMEM((2, CHW), jnp.int32),
        pltpu.VMEM((NCHW, CHW), jnp.int32),
        pltpu.VMEM((2, CHW, D), jnp.float32),
        pltpu.SemaphoreType.DMA,
        pltpu.SemaphoreType.DMA,
        pltpu.VMEM_SHARED((NPAD, D), jnp.float32),
    ],
)
def _row_agg_kernel(hs_hbm, src3_hbm, dst3_hbm, zero_hbm, out_hbm,
                    sring_v, didx_v, rows_v, sem_a, sem_b, acc_sh):
    cid = lax.axis_index("c")
    sid = lax.axis_index("s")
    wid = cid * NS + sid
    pltpu.sync_copy(zero_hbm.at[pl.ds(sid * RPT, RPT)],
                    acc_sh.at[pl.ds(sid * RPT, RPT)])
    pltpu.sync_copy(dst3_hbm.at[wid], didx_v)
    plsc.subcore_barrier()

    def g(b, sem):
        return pltpu.make_async_copy(hs_hbm.at[sring_v.at[b]],
                                     rows_v.at[b], sem)

    pltpu.sync_copy(src3_hbm.at[wid, 0], sring_v.at[0])
    g(0, sem_a).start()

    def body(p, c):
        c0 = 2 * p
        pltpu.sync_copy(src3_hbm.at[wid, c0 + 1], sring_v.at[1])
        g(1, sem_b).start()
        g(0, sem_a).wait()
        pltpu.sync_copy(rows_v.at[0], acc_sh.at[didx_v.at[c0]], add=True)

        @pl.when(p < NPAIR - 1)
        def _():
            pltpu.sync_copy(src3_hbm.at[wid, c0 + 2], sring_v.at[0])
            g(0, sem_a).start()

        g(1, sem_b).wait()
        pltpu.sync_copy(rows_v.at[1], acc_sh.at[didx_v.at[c0 + 1]], add=True)
        return c

    lax.fori_loop(0, NPAIR, body, 0)
    plsc.subcore_barrier()
    pltpu.sync_copy(acc_sh.at[pl.ds(sid * RPT, RPT)],
                    out_hbm.at[cid, pl.ds(sid * RPT, RPT)])


@functools.partial(
    pl.kernel,
    out_type=jax.ShapeDtypeStruct((NC, NPAD), jnp.float32),
    mesh=_mesh,
    compiler_params=_sc_params,
    scratch_types=[
        pltpu.VMEM((EPW,), jnp.int32),
        pltpu.VMEM((EPW,), jnp.int32),
        pltpu.VMEM((N,), jnp.float32),
        pltpu.VMEM((NPAD,), jnp.float32),
        pltpu.VMEM((RPT,), jnp.float32),
        pltpu.VMEM((RPT,), jnp.float32),
        pltpu.VMEM_SHARED((NS, NPAD), jnp.float32),
    ],
)
def _scalar_agg_kernel(ss_hbm, src_hbm, dst_hbm, out_hbm,
                       sidx_v, didx_v, ss_v, acc_v, red_v, tmp_v, shared):
    cid = lax.axis_index("c")
    sid = lax.axis_index("s")
    wid = cid * NS + sid
    _zero_ref(acc_v, NPAD)
    pltpu.sync_copy(ss_hbm, ss_v)
    pltpu.sync_copy(src_hbm.at[pl.ds(wid * EPW, EPW)], sidx_v)
    pltpu.sync_copy(dst_hbm.at[pl.ds(wid * EPW, EPW)], didx_v)

    def body(i, c):
        svec = sidx_v[pl.ds(i * L, L)]
        dvec = didx_v[pl.ds(i * L, L)]
        vals = plsc.load_gather(ss_v, [svec])
        plsc.addupdate_scatter(acc_v, [dvec], vals)
        return c

    lax.fori_loop(0, EPW // L, body, 0)
    pltpu.sync_copy(acc_v, shared.at[sid])
    plsc.subcore_barrier()
    _reduce_tiles(shared, red_v, tmp_v, out_hbm, cid, sid)


_BR = 1000


def _tc_hs_body(x_ref, w_ref, degp_ref, hs_ref):
    deg = degp_ref[0] + degp_ref[1] + 1.0
    dinv = lax.rsqrt(deg)
    h = jnp.dot(jnp.abs(x_ref[...]), w_ref[...],
                preferred_element_type=jnp.float32)
    hs_ref[...] = h * dinv


def _tc_mid_body(accp_ref, hs_ref, degp_ref, b1_ref, w2_ref, ss_ref):
    deg = degp_ref[0] + degp_ref[1] + 1.0
    dinv = lax.rsqrt(deg)
    tot = accp_ref[0] + accp_ref[1] + hs_ref[...]
    out1 = jnp.maximum(tot * dinv + b1_ref[...], 0.0)
    s = jnp.dot(out1, w2_ref[...], preferred_element_type=jnp.float32)
    ss_ref[...] = s * dinv


def _tc_out_body(qp_ref, ss_ref, degp_ref, b2_ref, o_ref):
    deg = degp_ref[0] + degp_ref[1] + 1.0
    dinv = lax.rsqrt(deg)
    out2 = (qp_ref[0] + qp_ref[1] + ss_ref[...]) * dinv + b2_ref[...]
    o_ref[...] = jax.nn.sigmoid(jnp.maximum(out2, 0.0))


def kernel(x, edge_index, W1, b1, W2, b2):
    src = edge_index[0].astype(jnp.int32)
    dst = edge_index[1].astype(jnp.int32)

    degp = _deg_kernel(dst)
    degp3 = degp[:, :N].reshape(NC, N, 1)

    hs = pl.pallas_call(
        _tc_hs_body,
        grid=(N // _BR,),
        in_specs=[
            pl.BlockSpec((_BR, D), lambda i: (i, 0)),
            pl.BlockSpec((D, D), lambda i: (0, 0)),
            pl.BlockSpec((NC, _BR, 1), lambda i: (0, i, 0)),
        ],
        out_specs=pl.BlockSpec((_BR, D), lambda i: (i, 0)),
        out_shape=jax.ShapeDtypeStruct((N, D), jnp.float32),
    )(x, W1, degp3)

    zrows = jnp.zeros((NPAD, D), jnp.float32)
    pad_i = jnp.arange(EPAD - E, dtype=jnp.int32)
    srcp = jnp.concatenate([src, pad_i % N])
    dstp = jnp.concatenate([dst, N + pad_i % (NPAD - N)])
    accp = _row_agg_kernel(hs, srcp.reshape(NW, NCHW, CHW),
                           dstp.reshape(NW, NCHW, CHW), zrows)
    accp_n = accp[:, :N, :]

    ss = pl.pallas_call(
        _tc_mid_body,
        grid=(N // _BR,),
        in_specs=[
            pl.BlockSpec((NC, _BR, D), lambda i: (0, i, 0)),
            pl.BlockSpec((_BR, D), lambda i: (i, 0)),
            pl.BlockSpec((NC, _BR, 1), lambda i: (0, i, 0)),
            pl.BlockSpec((1, D), lambda i: (0, 0)),
            pl.BlockSpec((D, 1), lambda i: (0, 0)),
        ],
        out_specs=pl.BlockSpec((_BR, 1), lambda i: (i, 0)),
        out_shape=jax.ShapeDtypeStruct((N, 1), jnp.float32),
    )(accp_n, hs, degp3, b1.reshape(1, D), W2)

    qp = _scalar_agg_kernel(ss.reshape(N), src, dst)
    qp3 = qp[:, :N].reshape(NC, N, 1)

    y = pl.pallas_call(
        _tc_out_body,
        grid=(1,),
        in_specs=[
            pl.BlockSpec((NC, N, 1), lambda i: (0, 0, 0)),
            pl.BlockSpec((N, 1), lambda i: (0, 0)),
            pl.BlockSpec((NC, N, 1), lambda i: (0, 0, 0)),
            pl.BlockSpec((1, 1), lambda i: (0, 0)),
        ],
        out_specs=pl.BlockSpec((N, 1), lambda i: (0, 0)),
        out_shape=jax.ShapeDtypeStruct((N, 1), jnp.float32),
    )(qp3, ss, degp3, b2.reshape(1, 1))

    return y


# speedup vs baseline: 2.5351x; 1.0011x over previous
import functools

import jax
import jax.numpy as jnp
from jax import lax
from jax.experimental import pallas as pl
from jax.experimental.pallas import tpu as pltpu
from jax.experimental.pallas import tpu_sc as plsc

N = 10000
E = 320000
D = 128
NC = 2
NS = 16
L = 16
NW = NC * NS
EPW = E // NW
NPAD = 10240
RPT = NPAD // NS
CHW = 128
NCHW = 80
NPAIR = NCHW // 2
EPAD = NW * NCHW * CHW

_mesh = plsc.VectorSubcoreMesh(core_axis_name="c", subcore_axis_name="s")
_sc_params = pltpu.CompilerParams(needs_layout_passes=False)


def _zero_ref(ref, nwords):
    z = jnp.zeros((L,), jnp.float32)

    def body(i, c):
        ref[pl.ds(i * L, L)] = z
        return c

    lax.fori_loop(0, nwords // L, body, 0)


def _reduce_tiles(shared, red_v, tmp_v, out_hbm, cid, sid):
    base = sid * RPT
    pltpu.sync_copy(shared.at[0, pl.ds(base, RPT)], red_v)

    def body(k, c):
        pltpu.sync_copy(shared.at[k, pl.ds(base, RPT)], tmp_v)

        def add(i, c2):
            red_v[pl.ds(i * L, L)] = red_v[pl.ds(i * L, L)] + tmp_v[pl.ds(i * L, L)]
            return c2

        lax.fori_loop(0, RPT // L, add, 0)
        return c

    lax.fori_loop(1, NS, body, 0)
    pltpu.sync_copy(red_v, out_hbm.at[cid, pl.ds(base, RPT)])


@functools.partial(
    pl.kernel,
    out_type=jax.ShapeDtypeStruct((NC, NPAD), jnp.float32),
    mesh=_mesh,
    compiler_params=_sc_params,
    scratch_types=[
        pltpu.VMEM((EPW,), jnp.int32),
        pltpu.VMEM((NPAD,), jnp.float32),
        pltpu.VMEM((RPT,), jnp.float32),
        pltpu.VMEM((RPT,), jnp.float32),
        pltpu.VMEM_SHARED((NS, NPAD), jnp.float32),
    ],
)
def _deg_kernel(dst_hbm, out_hbm, idx_v, acc_v, red_v, tmp_v, shared):
    cid = lax.axis_index("c")
    sid = lax.axis_index("s")
    wid = cid * NS + sid
    _zero_ref(acc_v, NPAD)
    pltpu.sync_copy(dst_hbm.at[pl.ds(wid * EPW, EPW)], idx_v)
    ones = jnp.ones((L,), jnp.float32)

    def body(i, c):
        dvec = idx_v[pl.ds(i * L, L)]
        plsc.addupdate_scatter(acc_v, [dvec], ones)
        return c

    lax.fori_loop(0, EPW // L, body, 0)
    pltpu.sync_copy(acc_v, shared.at[sid])
    plsc.subcore_barrier()
    _reduce_tiles(shared, red_v, tmp_v, out_hbm, cid, sid)


@functools.partial(
    pl.kernel,
    out_type=jax.ShapeDtypeStruct((NC, NPAD, D), jnp.float32),
    mesh=_mesh,
    compiler_params=_sc_params,
    scratch_types=[
        pltpu.VMEM((2, CHW), jnp.int32),
        pltpu.VMEM((NCHW, CHW), jnp.int32),
        pltpu.VMEM((2, CHW, D), jnp.float32),
        pltpu.SemaphoreType.DMA,
        pltpu.SemaphoreType.DMA,
        pltpu.SemaphoreType.DMA,
        pltpu.SemaphoreType.DMA,
        pltpu.VMEM_SHARED((NPAD, D), jnp.float32),
    ],
)
def _row_agg_kernel(hs_hbm, src3_hbm, dst3_hbm, zero_hbm, out_hbm,
                    sring_v, didx_v, rows_v, sem_a, sem_b, sem_sa, sem_sb,
                    acc_sh):
    cid = lax.axis_index("c")
    sid = lax.axis_index("s")
    wid = cid * NS + sid
    pltpu.sync_copy(zero_hbm.at[pl.ds(sid * RPT, RPT)],
                    acc_sh.at[pl.ds(sid * RPT, RPT)])
    pltpu.sync_copy(dst3_hbm.at[wid], didx_v)
    plsc.subcore_barrier()

    def g(b, sem):
        return pltpu.make_async_copy(hs_hbm.at[sring_v.at[b]],
                                     rows_v.at[b], sem)

    def s_start(b, ci, sem):
        pltpu.async_copy(rows_v.at[b], acc_sh.at[didx_v.at[ci]], sem,
                         add=True)

    def s_wait(b, ci, sem):
        pltpu.make_async_copy(rows_v.at[b], acc_sh.at[didx_v.at[ci]],
                              sem).wait()

    pltpu.sync_copy(src3_hbm.at[wid, 0], sring_v.at[0])
    g(0, sem_a).start()

    def body(p, c):
        c0 = 2 * p

        @pl.when(p > 0)
        def _():
            s_wait(1, c0 - 1, sem_sb)

        pltpu.sync_copy(src3_hbm.at[wid, c0 + 1], sring_v.at[1])
        g(1, sem_b).start()
        g(0, sem_a).wait()
        s_start(0, c0, sem_sa)

        @pl.when(p < NPAIR - 1)
        def _():
            pltpu.sync_copy(src3_hbm.at[wid, c0 + 2], sring_v.at[0])

        g(1, sem_b).wait()
        s_start(1, c0 + 1, sem_sb)

        @pl.when(p < NPAIR - 1)
        def _():
            s_wait(0, c0, sem_sa)
            g(0, sem_a).start()

        return c

    lax.fori_loop(0, NPAIR, body, 0)
    s_wait(0, NCHW - 2, sem_sa)
    s_wait(1, NCHW - 1, sem_sb)
    plsc.subcore_barrier()
    pltpu.sync_copy(acc_sh.at[pl.ds(sid * RPT, RPT)],
                    out_hbm.at[cid, pl.ds(sid * RPT, RPT)])


@functools.partial(
    pl.kernel,
    out_type=jax.ShapeDtypeStruct((NC, NPAD), jnp.float32),
    mesh=_mesh,
    compiler_params=_sc_params,
    scratch_types=[
        pltpu.VMEM((EPW,), jnp.int32),
        pltpu.VMEM((EPW,), jnp.int32),
        pltpu.VMEM((N,), jnp.float32),
        pltpu.VMEM((NPAD,), jnp.float32),
        pltpu.VMEM((RPT,), jnp.float32),
        pltpu.VMEM((RPT,), jnp.float32),
        pltpu.VMEM_SHARED((NS, NPAD), jnp.float32),
    ],
)
def _scalar_agg_kernel(ss_hbm, src_hbm, dst_hbm, out_hbm,
                       sidx_v, didx_v, ss_v, acc_v, red_v, tmp_v, shared):
    cid = lax.axis_index("c")
    sid = lax.axis_index("s")
    wid = cid * NS + sid
    _zero_ref(acc_v, NPAD)
    pltpu.sync_copy(ss_hbm, ss_v)
    pltpu.sync_copy(src_hbm.at[pl.ds(wid * EPW, EPW)], sidx_v)
    pltpu.sync_copy(dst_hbm.at[pl.ds(wid * EPW, EPW)], didx_v)

    def body(i, c):
        svec = sidx_v[pl.ds(i * L, L)]
        dvec = didx_v[pl.ds(i * L, L)]
        vals = plsc.load_gather(ss_v, [svec])
        plsc.addupdate_scatter(acc_v, [dvec], vals)
        return c

    lax.fori_loop(0, EPW // L, body, 0)
    pltpu.sync_copy(acc_v, shared.at[sid])
    plsc.subcore_barrier()
    _reduce_tiles(shared, red_v, tmp_v, out_hbm, cid, sid)


_BR = 1000


def _tc_hs_body(x_ref, w_ref, degp_ref, hs_ref):
    deg = degp_ref[0] + degp_ref[1] + 1.0
    dinv = lax.rsqrt(deg)
    h = jnp.dot(jnp.abs(x_ref[...]), w_ref[...],
                preferred_element_type=jnp.float32)
    hs_ref[...] = h * dinv


def _tc_mid_body(accp_ref, hs_ref, degp_ref, b1_ref, w2_ref, ss_ref):
    deg = degp_ref[0] + degp_ref[1] + 1.0
    dinv = lax.rsqrt(deg)
    tot = accp_ref[0] + accp_ref[1] + hs_ref[...]
    out1 = jnp.maximum(tot * dinv + b1_ref[...], 0.0)
    s = jnp.dot(out1, w2_ref[...], preferred_element_type=jnp.float32)
    ss_ref[...] = s * dinv


def _tc_out_body(qp_ref, ss_ref, degp_ref, b2_ref, o_ref):
    deg = degp_ref[0] + degp_ref[1] + 1.0
    dinv = lax.rsqrt(deg)
    out2 = (qp_ref[0] + qp_ref[1] + ss_ref[...]) * dinv + b2_ref[...]
    o_ref[...] = jax.nn.sigmoid(jnp.maximum(out2, 0.0))


def kernel(x, edge_index, W1, b1, W2, b2):
    src = edge_index[0].astype(jnp.int32)
    dst = edge_index[1].astype(jnp.int32)

    degp = _deg_kernel(dst)
    degp3 = degp[:, :N].reshape(NC, N, 1)

    hs = pl.pallas_call(
        _tc_hs_body,
        grid=(N // _BR,),
        in_specs=[
            pl.BlockSpec((_BR, D), lambda i: (i, 0)),
            pl.BlockSpec((D, D), lambda i: (0, 0)),
            pl.BlockSpec((NC, _BR, 1), lambda i: (0, i, 0)),
        ],
        out_specs=pl.BlockSpec((_BR, D), lambda i: (i, 0)),
        out_shape=jax.ShapeDtypeStruct((N, D), jnp.float32),
    )(x, W1, degp3)

    zrows = jnp.zeros((NPAD, D), jnp.float32)
    pad_i = jnp.arange(EPAD - E, dtype=jnp.int32)
    srcp = jnp.concatenate([src, pad_i % N])
    dstp = jnp.concatenate([dst, N + pad_i % (NPAD - N)])
    accp = _row_agg_kernel(hs, srcp.reshape(NW, NCHW, CHW),
                           dstp.reshape(NW, NCHW, CHW), zrows)
    accp_n = accp[:, :N, :]

    ss = pl.pallas_call(
        _tc_mid_body,
        grid=(N // _BR,),
        in_specs=[
            pl.BlockSpec((NC, _BR, D), lambda i: (0, i, 0)),
            pl.BlockSpec((_BR, D), lambda i: (i, 0)),
            pl.BlockSpec((NC, _BR, 1), lambda i: (0, i, 0)),
            pl.BlockSpec((1, D), lambda i: (0, 0)),
            pl.BlockSpec((D, 1), lambda i: (0, 0)),
        ],
        out_specs=pl.BlockSpec((_BR, 1), lambda i: (i, 0)),
        out_shape=jax.ShapeDtypeStruct((N, 1), jnp.float32),
    )(accp_n, hs, degp3, b1.reshape(1, D), W2)

    qp = _scalar_agg_kernel(ss.reshape(N), src, dst)
    qp3 = qp[:, :N].reshape(NC, N, 1)

    y = pl.pallas_call(
        _tc_out_body,
        grid=(1,),
        in_specs=[
            pl.BlockSpec((NC, N, 1), lambda i: (0, 0, 0)),
            pl.BlockSpec((N, 1), lambda i: (0, 0)),
            pl.BlockSpec((NC, N, 1), lambda i: (0, 0, 0)),
            pl.BlockSpec((1, 1), lambda i: (0, 0)),
        ],
        out_specs=pl.BlockSpec((N, 1), lambda i: (0, 0)),
        out_shape=jax.ShapeDtypeStruct((N, 1), jnp.float32),
    )(qp3, ss, degp3, b2.reshape(1, 1))

    return y


# speedup vs baseline: 2.6846x; 1.0590x over previous
import functools

import jax
import jax.numpy as jnp
from jax import lax
from jax.experimental import pallas as pl
from jax.experimental.pallas import tpu as pltpu
from jax.experimental.pallas import tpu_sc as plsc

N = 10000
E = 320000
D = 128
NC = 2
NS = 16
L = 16
NW = NC * NS
EPW = E // NW
NPAD = 10240
RPT = NPAD // NS
CHW = 128
NCHW = 80
NPAIR = NCHW // 2
EPAD = NW * NCHW * CHW

_mesh = plsc.VectorSubcoreMesh(core_axis_name="c", subcore_axis_name="s")
_sc_params = pltpu.CompilerParams(needs_layout_passes=False)


def _zero_ref(ref, nwords):
    z = jnp.zeros((L,), jnp.float32)

    def body(i, c):
        ref[pl.ds(i * L, L)] = z
        return c

    lax.fori_loop(0, nwords // L, body, 0)


def _reduce_tiles(shared, tmp2_v, red_v, sid):
    pltpu.sync_copy(shared.at[:, pl.ds(sid * RPT, RPT)], tmp2_v)

    def body(i, c):
        o = i * L
        acc16 = tmp2_v[0, pl.ds(o, L)]
        for k in range(1, NS):
            acc16 = acc16 + tmp2_v[k, pl.ds(o, L)]
        red_v[pl.ds(o, L)] = acc16
        return c

    lax.fori_loop(0, RPT // L, body, 0)


@functools.partial(
    pl.kernel,
    out_type=jax.ShapeDtypeStruct((NC, NPAD), jnp.float32),
    mesh=_mesh,
    compiler_params=_sc_params,
    scratch_types=[
        pltpu.VMEM((EPW,), jnp.int32),
        pltpu.VMEM((NPAD,), jnp.float32),
        pltpu.VMEM((RPT,), jnp.float32),
        pltpu.VMEM((NS, RPT), jnp.float32),
        pltpu.VMEM_SHARED((NS, NPAD), jnp.float32),
    ],
)
def _deg_kernel(dst_hbm, out_hbm, idx_v, acc_v, red_v, tmp2_v, shared):
    cid = lax.axis_index("c")
    sid = lax.axis_index("s")
    wid = cid * NS + sid
    _zero_ref(acc_v, NPAD)
    pltpu.sync_copy(dst_hbm.at[pl.ds(wid * EPW, EPW)], idx_v)
    ones = jnp.ones((L,), jnp.float32)

    def body(i, c):
        dvec = idx_v[pl.ds(i * L, L)]
        plsc.addupdate_scatter(acc_v, [dvec], ones)
        return c

    lax.fori_loop(0, EPW // L, body, 0)
    pltpu.sync_copy(acc_v, shared.at[sid])
    plsc.subcore_barrier()
    _reduce_tiles(shared, tmp2_v, red_v, sid)
    pltpu.sync_copy(red_v, out_hbm.at[cid, pl.ds(sid * RPT, RPT)])


@functools.partial(
    pl.kernel,
    out_type=jax.ShapeDtypeStruct((NC, NPAD, D), jnp.float32),
    mesh=_mesh,
    compiler_params=_sc_params,
    scratch_types=[
        pltpu.VMEM((2, CHW), jnp.int32),
        pltpu.VMEM((NCHW, CHW), jnp.int32),
        pltpu.VMEM((2, CHW, D), jnp.float32),
        pltpu.SemaphoreType.DMA,
        pltpu.SemaphoreType.DMA,
        pltpu.SemaphoreType.DMA,
        pltpu.SemaphoreType.DMA,
        pltpu.VMEM_SHARED((NPAD, D), jnp.float32),
    ],
)
def _row_agg_kernel(hs_hbm, src3_hbm, dst3_hbm, out_hbm,
                    sring_v, didx_v, rows_v, sem_a, sem_b, sem_sa, sem_sb,
                    acc_sh):
    cid = lax.axis_index("c")
    sid = lax.axis_index("s")
    wid = cid * NS + sid
    z = jnp.zeros((L,), jnp.float32)

    def zbody(j, c):
        rows_v[0, j // (D // L), pl.ds((j % (D // L)) * L, L)] = z
        return c

    lax.fori_loop(0, CHW * (D // L), zbody, 0)
    for k in range(RPT // CHW):
        pltpu.sync_copy(rows_v.at[0],
                        acc_sh.at[pl.ds(sid * RPT + k * CHW, CHW)])
    pltpu.sync_copy(dst3_hbm.at[wid], didx_v)
    plsc.subcore_barrier()

    def g(b, sem):
        return pltpu.make_async_copy(hs_hbm.at[sring_v.at[b]],
                                     rows_v.at[b], sem)

    def s_start(b, ci, sem):
        pltpu.async_copy(rows_v.at[b], acc_sh.at[didx_v.at[ci]], sem,
                         add=True)

    def s_wait(b, ci, sem):
        pltpu.make_async_copy(rows_v.at[b], acc_sh.at[didx_v.at[ci]],
                              sem).wait()

    pltpu.sync_copy(src3_hbm.at[wid, 0], sring_v.at[0])
    g(0, sem_a).start()

    def body(p, c):
        c0 = 2 * p

        @pl.when(p > 0)
        def _():
            s_wait(1, c0 - 1, sem_sb)

        pltpu.sync_copy(src3_hbm.at[wid, c0 + 1], sring_v.at[1])
        g(1, sem_b).start()
        g(0, sem_a).wait()
        s_start(0, c0, sem_sa)

        @pl.when(p < NPAIR - 1)
        def _():
            pltpu.sync_copy(src3_hbm.at[wid, c0 + 2], sring_v.at[0])

        g(1, sem_b).wait()
        s_start(1, c0 + 1, sem_sb)

        @pl.when(p < NPAIR - 1)
        def _():
            s_wait(0, c0, sem_sa)
            g(0, sem_a).start()

        return c

    lax.fori_loop(0, NPAIR, body, 0)
    s_wait(0, NCHW - 2, sem_sa)
    s_wait(1, NCHW - 1, sem_sb)
    plsc.subcore_barrier()
    pltpu.sync_copy(acc_sh.at[pl.ds(sid * RPT, RPT)],
                    out_hbm.at[cid, pl.ds(sid * RPT, RPT)])


@functools.partial(
    pl.kernel,
    out_type=jax.ShapeDtypeStruct((NC, NPAD), jnp.float32),
    mesh=_mesh,
    compiler_params=_sc_params,
    scratch_types=[
        pltpu.VMEM((EPW,), jnp.int32),
        pltpu.VMEM((EPW,), jnp.int32),
        pltpu.VMEM((NPAD,), jnp.float32),
        pltpu.VMEM((NPAD,), jnp.float32),
        pltpu.VMEM((RPT,), jnp.float32),
        pltpu.VMEM((NS, RPT), jnp.float32),
        pltpu.VMEM_SHARED((NS, NPAD), jnp.float32),
    ],
)
def _scalar_agg_kernel(ss_hbm, src_hbm, dst_hbm, out_hbm,
                       sidx_v, didx_v, ss_v, acc_v, red_v, tmp2_v, shared):
    cid = lax.axis_index("c")
    sid = lax.axis_index("s")
    wid = cid * NS + sid
    _zero_ref(acc_v, NPAD)
    pltpu.sync_copy(ss_hbm, ss_v.at[pl.ds(0, N)])
    pltpu.sync_copy(src_hbm.at[pl.ds(wid * EPW, EPW)], sidx_v)
    pltpu.sync_copy(dst_hbm.at[pl.ds(wid * EPW, EPW)], didx_v)

    def body(i, c):
        svec = sidx_v[pl.ds(i * L, L)]
        dvec = didx_v[pl.ds(i * L, L)]
        vals = plsc.load_gather(ss_v, [svec])
        plsc.addupdate_scatter(acc_v, [dvec], vals)
        return c

    lax.fori_loop(0, EPW // L, body, 0)
    pltpu.sync_copy(acc_v, shared.at[sid])
    plsc.subcore_barrier()
    _reduce_tiles(shared, tmp2_v, red_v, sid)
    pltpu.sync_copy(red_v, out_hbm.at[cid, pl.ds(sid * RPT, RPT)])


_BR = 1000


def _tc_hs_body(x_ref, w_ref, degp_ref, hs_ref):
    deg = degp_ref[0] + degp_ref[1] + 1.0
    dinv = lax.rsqrt(deg)
    h = jnp.dot(jnp.abs(x_ref[...]), w_ref[...],
                preferred_element_type=jnp.float32)
    hs_ref[...] = h * dinv


def _tc_mid_body(accp_ref, hs_ref, degp_ref, b1_ref, w2_ref, ss_ref):
    deg = degp_ref[0] + degp_ref[1] + 1.0
    dinv = lax.rsqrt(deg)
    tot = accp_ref[0] + accp_ref[1] + hs_ref[...]
    out1 = jnp.maximum(tot * dinv + b1_ref[...], 0.0)
    s = jnp.dot(out1, w2_ref[...], preferred_element_type=jnp.float32)
    ss_ref[...] = s * dinv


def _tc_out_body(qp_ref, ss_ref, degp_ref, b2_ref, o_ref):
    deg = degp_ref[0] + degp_ref[1] + 1.0
    dinv = lax.rsqrt(deg)
    out2 = (qp_ref[0] + qp_ref[1] + ss_ref[...]) * dinv + b2_ref[...]
    o_ref[...] = jax.nn.sigmoid(jnp.maximum(out2, 0.0))


def kernel(x, edge_index, W1, b1, W2, b2):
    src = edge_index[0].astype(jnp.int32)
    dst = edge_index[1].astype(jnp.int32)

    degp = _deg_kernel(dst)
    degp3 = degp.reshape(NC, NPAD, 1)

    hs = pl.pallas_call(
        _tc_hs_body,
        grid=(N // _BR,),
        in_specs=[
            pl.BlockSpec((_BR, D), lambda i: (i, 0)),
            pl.BlockSpec((D, D), lambda i: (0, 0)),
            pl.BlockSpec((NC, _BR, 1), lambda i: (0, i, 0)),
        ],
        out_specs=pl.BlockSpec((_BR, D), lambda i: (i, 0)),
        out_shape=jax.ShapeDtypeStruct((N, D), jnp.float32),
    )(x, W1, degp3)

    pad_i = jnp.arange(EPAD - E, dtype=jnp.int32)
    srcp = jnp.concatenate([src, pad_i % N])
    dstp = jnp.concatenate([dst, N + pad_i % (NPAD - N)])
    accp = _row_agg_kernel(hs, srcp.reshape(NW, NCHW, CHW),
                           dstp.reshape(NW, NCHW, CHW))

    ss = pl.pallas_call(
        _tc_mid_body,
        grid=(N // _BR,),
        in_specs=[
            pl.BlockSpec((NC, _BR, D), lambda i: (0, i, 0)),
            pl.BlockSpec((_BR, D), lambda i: (i, 0)),
            pl.BlockSpec((NC, _BR, 1), lambda i: (0, i, 0)),
            pl.BlockSpec((1, D), lambda i: (0, 0)),
            pl.BlockSpec((D, 1), lambda i: (0, 0)),
        ],
        out_specs=pl.BlockSpec((_BR, 1), lambda i: (i, 0)),
        out_shape=jax.ShapeDtypeStruct((N, 1), jnp.float32),
    )(accp, hs, degp3, b1.reshape(1, D), W2)

    qp = _scalar_agg_kernel(ss.reshape(N), src, dst)
    qp3 = qp.reshape(NC, NPAD, 1)

    y = pl.pallas_call(
        _tc_out_body,
        grid=(1,),
        in_specs=[
            pl.BlockSpec((NC, N, 1), lambda i: (0, 0, 0)),
            pl.BlockSpec((N, 1), lambda i: (0, 0)),
            pl.BlockSpec((NC, N, 1), lambda i: (0, 0, 0)),
            pl.BlockSpec((1, 1), lambda i: (0, 0)),
        ],
        out_specs=pl.BlockSpec((N, 1), lambda i: (0, 0)),
        out_shape=jax.ShapeDtypeStruct((N, 1), jnp.float32),
    )(qp3, ss, degp3, b2.reshape(1, 1))

    return y
